# baseline (speedup 1.0000x reference)
import jax
import jax.numpy as jnp
from jax.experimental import pallas as pl
from jax.experimental.pallas import tpu as pltpu

N = 10000
NFEAT = 128
NHID = 64
NCLASS = 16

BM = 200
NM = N // BM

_DOT = (((1,), (0,)), ((), ()))


def _dot(a, b):
    return jax.lax.dot_general(a, b, _DOT, preferred_element_type=jnp.float32)


def _attention(ol, oh, om, avs_ref, av_ref):
    sl = jax.nn.sigmoid(jnp.sum(ol * avs_ref[0:1, :], axis=1, keepdims=True))
    sh = jax.nn.sigmoid(jnp.sum(oh * avs_ref[1:2, :], axis=1, keepdims=True))
    sm = jax.nn.sigmoid(jnp.sum(om * avs_ref[2:3, :], axis=1, keepdims=True))
    logits = [
        (sl * av_ref[0, j] + sh * av_ref[1, j] + sm * av_ref[2, j]) * (1.0 / 3.0)
        for j in range(3)
    ]
    mx = jnp.maximum(jnp.maximum(logits[0], logits[1]), logits[2])
    e0 = jnp.exp(logits[0] - mx)
    e1 = jnp.exp(logits[1] - mx)
    e2 = jnp.exp(logits[2] - mx)
    inv = 1.0 / (e0 + e1 + e2)
    return e0 * inv, e1 * inv, e2 * inv


def _proj_kernel(x_ref, wl_ref, wh_ref, xl_ref, xh_ref):
    x = x_ref[...]
    xl_ref[...] = _dot(x, wl_ref[...])
    xh_ref[...] = _dot(x, wh_ref[...])


def _l1_kernel(adjl_ref, adjh_ref, xl_ref, xh_ref, x_ref, wm_ref, wl2_ref,
               wh2_ref, avs_ref, av_ref, h_ref, hl_ref, hh_ref):
    ol = jnp.maximum(_dot(adjl_ref[...], xl_ref[...]), 0.0)
    oh = jnp.maximum(_dot(adjh_ref[...], xh_ref[...]), 0.0)
    om = jnp.maximum(_dot(x_ref[...], wm_ref[...]), 0.0)
    al, ah, am = _attention(ol, oh, om, avs_ref, av_ref)
    h = 3.0 * (al * ol + ah * oh + am * om)
    h_ref[...] = h
    hl_ref[...] = _dot(h, wl2_ref[...])
    hh_ref[...] = _dot(h, wh2_ref[...])


def _l2_kernel(adjl_ref, adjh_ref, hl_ref, hh_ref, h_ref, wm2_ref,
               avs2_ref, av2_ref, out_ref):
    ol = jnp.maximum(_dot(adjl_ref[...], hl_ref[...]), 0.0)
    oh = jnp.maximum(_dot(adjh_ref[...], hh_ref[...]), 0.0)
    om = jnp.maximum(_dot(h_ref[...], wm2_ref[...]), 0.0)
    al, ah, am = _attention(ol, oh, om, avs2_ref, av2_ref)
    out = 3.0 * (al * ol + ah * oh + am * om)
    z = out - jnp.max(out, axis=1, keepdims=True)
    out_ref[...] = z - jnp.log(jnp.sum(jnp.exp(z), axis=1, keepdims=True))


def kernel(x, adj_low, adj_high, weight_low, weight_high, weight_mlp,
           att_vec_low, att_vec_high, att_vec_mlp, att_vec, weight_low2,
           weight_high2, weight_mlp2, att_vec_low2, att_vec_high2,
           att_vec_mlp2, att_vec2):
    xl, xh = pl.pallas_call(
        _proj_kernel,
        out_shape=[jax.ShapeDtypeStruct((N, NHID), jnp.float32)] * 2,
    )(x, weight_low, weight_high)

    avs = jnp.concatenate(
        [att_vec_low.T, att_vec_high.T, att_vec_mlp.T], axis=0)
    avs2 = jnp.concatenate(
        [att_vec_low2.T, att_vec_high2.T, att_vec_mlp2.T], axis=0)

    grid = (NM,)
    adj_spec = pl.BlockSpec((BM, N), lambda m: (m, 0))
    row_spec = lambda w: pl.BlockSpec((BM, w), lambda m: (m, 0))
    full_spec = lambda a, b: pl.BlockSpec((a, b), lambda m: (0, 0))
    cparams = pltpu.CompilerParams(dimension_semantics=("arbitrary",))

    h, hl, hh = pl.pallas_call(
        _l1_kernel,
        grid=grid,
        in_specs=[
            adj_spec,
            adj_spec,
            full_spec(N, NHID),
            full_spec(N, NHID),
            row_spec(NFEAT),
            full_spec(NFEAT, NHID),
            full_spec(NHID, NCLASS),
            full_spec(NHID, NCLASS),
            full_spec(3, NHID),
            full_spec(3, 3),
        ],
        out_specs=[
            row_spec(NHID),
            row_spec(NCLASS),
            row_spec(NCLASS),
        ],
        out_shape=[
            jax.ShapeDtypeStruct((N, NHID), jnp.float32),
            jax.ShapeDtypeStruct((N, NCLASS), jnp.float32),
            jax.ShapeDtypeStruct((N, NCLASS), jnp.float32),
        ],
        compiler_params=cparams,
    )(adj_low, adj_high, xl, xh, x, weight_mlp, weight_low2, weight_high2,
      avs, att_vec)

    out = pl.pallas_call(
        _l2_kernel,
        grid=grid,
        in_specs=[
            adj_spec,
            adj_spec,
            full_spec(N, NCLASS),
            full_spec(N, NCLASS),
            row_spec(NHID),
            full_spec(NHID, NCLASS),
            full_spec(3, NCLASS),
            full_spec(3, 3),
        ],
        out_specs=row_spec(NCLASS),
        out_shape=jax.ShapeDtypeStruct((N, NCLASS), jnp.float32),
        compiler_params=cparams,
    )(adj_low, adj_high, hl, hh, h, weight_mlp2, avs2, att_vec2)

    return out


# speedup vs baseline: 1.0359x; 1.0359x over previous
import jax
import jax.numpy as jnp
from jax.experimental import pallas as pl
from jax.experimental.pallas import tpu as pltpu

N = 10000
NFEAT = 128
NHID = 64
NCLASS = 16

BM = 200
NM = N // BM

_DOT = (((1,), (0,)), ((), ()))


def _dot(a, b):
    return jax.lax.dot_general(a, b, _DOT, preferred_element_type=jnp.float32)


def _attention(ol, oh, om, avs_ref, av_ref):
    sl = jax.nn.sigmoid(jnp.sum(ol * avs_ref[0:1, :], axis=1, keepdims=True))
    sh = jax.nn.sigmoid(jnp.sum(oh * avs_ref[1:2, :], axis=1, keepdims=True))
    sm = jax.nn.sigmoid(jnp.sum(om * avs_ref[2:3, :], axis=1, keepdims=True))
    logits = [
        (sl * av_ref[0, j] + sh * av_ref[1, j] + sm * av_ref[2, j]) * (1.0 / 3.0)
        for j in range(3)
    ]
    mx = jnp.maximum(jnp.maximum(logits[0], logits[1]), logits[2])
    e0 = jnp.exp(logits[0] - mx)
    e1 = jnp.exp(logits[1] - mx)
    e2 = jnp.exp(logits[2] - mx)
    inv = 1.0 / (e0 + e1 + e2)
    return e0 * inv, e1 * inv, e2 * inv


def _fused_kernel(adjl_ref, adjh_ref, x_ref, wl_ref, wh_ref, wm_ref,
                  wl2_ref, wh2_ref, wm2_ref, avs_ref, av_ref, avs2_ref,
                  av2_ref, out_ref, xl_s, xh_s, hl_s, hh_s, hm_s):
    layer = pl.program_id(0)
    m = pl.program_id(1)
    rows = pl.ds(m * BM, BM)

    @pl.when(jnp.logical_and(layer == 0, m == 0))
    def _():
        xf = x_ref[...]
        xl_s[...] = _dot(xf, wl_ref[...])
        xh_s[...] = _dot(xf, wh_ref[...])

    @pl.when(layer == 0)
    def _():
        ol = jnp.maximum(_dot(adjl_ref[...], xl_s[...]), 0.0)
        oh = jnp.maximum(_dot(adjh_ref[...], xh_s[...]), 0.0)
        om = jnp.maximum(_dot(x_ref[rows, :], wm_ref[...]), 0.0)
        al, ah, am = _attention(ol, oh, om, avs_ref, av_ref)
        h = 3.0 * (al * ol + ah * oh + am * om)
        hl_s[rows, :] = _dot(h, wl2_ref[...])
        hh_s[rows, :] = _dot(h, wh2_ref[...])
        hm_s[rows, :] = jnp.maximum(_dot(h, wm2_ref[...]), 0.0)

    @pl.when(layer == 1)
    def _():
        ol = jnp.maximum(_dot(adjl_ref[...], hl_s[...]), 0.0)
        oh = jnp.maximum(_dot(adjh_ref[...], hh_s[...]), 0.0)
        om = hm_s[rows, :]
        al, ah, am = _attention(ol, oh, om, avs2_ref, av2_ref)
        o = 3.0 * (al * ol + ah * oh + am * om)
        z = o - jnp.max(o, axis=1, keepdims=True)
        out_ref[...] = z - jnp.log(jnp.sum(jnp.exp(z), axis=1, keepdims=True))


def kernel(x, adj_low, adj_high, weight_low, weight_high, weight_mlp,
           att_vec_low, att_vec_high, att_vec_mlp, att_vec, weight_low2,
           weight_high2, weight_mlp2, att_vec_low2, att_vec_high2,
           att_vec_mlp2, att_vec2):
    avs = jnp.concatenate(
        [att_vec_low.T, att_vec_high.T, att_vec_mlp.T], axis=0)
    avs2 = jnp.concatenate(
        [att_vec_low2.T, att_vec_high2.T, att_vec_mlp2.T], axis=0)

    adj_spec = pl.BlockSpec((BM, N), lambda l, m: (m, 0))
    full_spec = lambda a, b: pl.BlockSpec((a, b), lambda l, m: (0, 0))

    out = pl.pallas_call(
        _fused_kernel,
        grid=(2, NM),
        in_specs=[
            adj_spec,
            adj_spec,
            full_spec(N, NFEAT),
            full_spec(NFEAT, NHID),
            full_spec(NFEAT, NHID),
            full_spec(NFEAT, NHID),
            full_spec(NHID, NCLASS),
            full_spec(NHID, NCLASS),
            full_spec(NHID, NCLASS),
            full_spec(3, NHID),
            full_spec(3, 3),
            full_spec(3, NCLASS),
            full_spec(3, 3),
        ],
        out_specs=pl.BlockSpec((BM, NCLASS), lambda l, m: (m, 0)),
        out_shape=jax.ShapeDtypeStruct((N, NCLASS), jnp.float32),
        scratch_shapes=[
            pltpu.VMEM((N, NHID), jnp.float32),
            pltpu.VMEM((N, NHID), jnp.float32),
            pltpu.VMEM((N, NCLASS), jnp.float32),
            pltpu.VMEM((N, NCLASS), jnp.float32),
            pltpu.VMEM((N, NCLASS), jnp.float32),
        ],
        compiler_params=pltpu.CompilerParams(
            dimension_semantics=("arbitrary", "arbitrary"),
            vmem_limit_bytes=100 * 1024 * 1024),
    )(adj_low, adj_high, x, weight_low, weight_high, weight_mlp,
      weight_low2, weight_high2, weight_mlp2, avs, att_vec, avs2, att_vec2)

    return out
